# baseline (speedup 1.0000x reference)
import functools

import jax
import jax.numpy as jnp
from jax import lax
from jax.experimental import pallas as pl
from jax.experimental.pallas import tpu as pltpu
from jax.experimental.pallas import tpu_sc as plsc

D = 300
DP = 384
HS = 128
ND = 10000
NDP = 10240
DSTP = 12288
EW = 160000
EWP = 163840
ED = 16384
B = 1024
L = 200
C = 20
VW = 100000
SCALE = float(D) ** -0.5

NS = 16
NCW = 40
NCD = 8
NCX = 56
EDP = NCD * 128 * 32
XFP = NCX * 128 * 32
CMAX = max(NCW, NCD, NCX) * 128

_MESH = plsc.VectorSubcoreMesh(core_axis_name="c", subcore_axis_name="s")


def _sc_a_body(wt0, wt1, wt2, dt0, dt1, dt2, zrows, izeros, dstn,
               sn_dis, sn_pmi, sn_top,
               si_dis, di_dis, si_pmi, di_pmi, si_top, di_top,
               snd_dis, snd_pmi, snd_top,
               sid_dis, did_dis, sid_pmi, did_pmi, sid_top, did_top,
               sums_w, dstf, sums_d,
               acc, nidc, didxc, didx128, b0, b1, m0, m1):
    cid = lax.axis_index("c")
    sid = lax.axis_index("s")
    wts = (wt0, wt1, wt2)
    dts = (dt0, dt1, dt2)
    bufs = (b0, b1)
    sems = (m0, m1)

    def zero_acc(rpt):
        pltpu.sync_copy(zrows.at[pl.ds(0, rpt)], acc.at[pl.ds(sid * rpt, rpt)])

    def stage_didx(j):
        for k in range(8):
            didx128[pl.ds(k * 16, 16)] = didxc[pl.ds(j * 128 + k * 16, 16)]

    def compose(sn, si1d, di1d, gbase, n):
        pltpu.sync_copy(si1d.at[pl.ds(gbase, n * 128)],
                        didxc.at[pl.ds(0, n * 128)])
        pltpu.sync_copy(izeros, nidc.at[pl.ds(n * 128, 384)])
        descs = [pltpu.async_copy(sn.at[didxc.at[pl.ds(j * 128, 128)]],
                                  nidc.at[pl.ds(j * 128, 128)], m0)
                 for j in range(n)]
        for d_ in descs:
            d_.wait()
        pltpu.sync_copy(di1d.at[pl.ds(gbase, n * 128)],
                        didxc.at[pl.ds(0, n * 128)])

    def load_direct(g1d, d1d, gbase, n):
        pltpu.sync_copy(g1d.at[pl.ds(gbase, n * 128)],
                        nidc.at[pl.ds(0, n * 128)])
        pltpu.sync_copy(d1d.at[pl.ds(gbase, n * 128)],
                        didxc.at[pl.ds(0, n * 128)])
        pltpu.sync_copy(izeros, nidc.at[pl.ds(n * 128, 384)])

    def pipe_scatter(tbl, n):
        def gref(j):
            return tbl.at[nidc.at[pl.ds(j * 128, 128)]]

        pltpu.async_copy(gref(0), b0, m0)
        pltpu.async_copy(gref(1), b1, m1)

        def body(j2, _):
            for k in range(2):
                j = j2 * 2 + k
                pltpu.make_async_copy(gref(j), bufs[k], sems[k]).wait()
                stage_didx(j)
                pltpu.sync_copy(bufs[k], acc.at[didx128], add=True)
                pltpu.async_copy(gref(j + 2), bufs[k], sems[k])
            return 0

        lax.fori_loop(0, n // 2, body, 0)
        for k in range(2):
            pltpu.make_async_copy(gref(0), bufs[k], sems[k]).wait()

    def finish(out_ref, r, s_, rpt):
        plsc.subcore_barrier()
        pltpu.sync_copy(acc.at[pl.ds(sid * rpt, rpt)],
                        out_ref.at[r, s_, cid, pl.ds(sid * rpt, rpt)])
        zero_acc(rpt)
        plsc.subcore_barrier()

    zero_acc(NDP // NS)
    plsc.subcore_barrier()

    word = ((sn_dis, si_dis, di_dis), (sn_pmi, si_pmi, di_pmi),
            (sn_top, si_top, di_top))
    for r, (sn, si1d, di1d) in enumerate(word):
        gbase = cid * (NCW * 128 * NS) + sid * (NCW * 128)
        compose(sn, si1d, di1d, gbase, NCW)
        for s_ in range(3):
            pipe_scatter(wts[s_], NCW)
            finish(sums_w, r, s_, NDP // NS)

    @pl.when(sid < 6)
    def _():
        w = cid * 6 + sid
        pltpu.sync_copy(dstn.at[pl.ds(w * 1024, 1024)],
                        nidc.at[pl.ds(0, 1024)])
        for j in range(8):
            for s_ in range(3):
                pltpu.async_copy(
                    wts[s_].at[nidc.at[pl.ds(j * 128, 128)]], b0, m0
                ).wait()
                pltpu.sync_copy(b0, dstf.at[s_, pl.ds(w * 1024 + j * 128, 128)])

    docr = ((snd_dis, sid_dis, did_dis), (snd_pmi, sid_pmi, did_pmi),
            (snd_top, sid_top, did_top))
    for r, (sn, si1d, di1d) in enumerate(docr):
        gbase = cid * (NCD * 128 * NS) + sid * (NCD * 128)
        compose(sn, si1d, di1d, gbase, NCD)
        for s_ in range(3):
            pipe_scatter(dts[s_], NCD)
            finish(sums_d, r, s_, B // NS)


def _sc_a(wts, dts, zrows, izeros, dstn, word_idx, doc_idx):
    f = pl.kernel(
        _sc_a_body,
        out_type=[
            jax.ShapeDtypeStruct((3, 3, 2, NDP, HS), jnp.float32),
            jax.ShapeDtypeStruct((3, DSTP, HS), jnp.float32),
            jax.ShapeDtypeStruct((3, 3, 2, B, HS), jnp.float32),
        ],
        mesh=_MESH,
        scratch_types=[
            pltpu.VMEM_SHARED((NDP, HS), jnp.float32),
            pltpu.VMEM((NCW * 128 + 384,), jnp.int32),
            pltpu.VMEM((NCW * 128,), jnp.int32),
            pltpu.VMEM((128,), jnp.int32),
            pltpu.VMEM((HS, HS), jnp.float32),
            pltpu.VMEM((HS, HS), jnp.float32),
            pltpu.SemaphoreType.DMA,
            pltpu.SemaphoreType.DMA,
        ],
    )
    return f(*wts, *dts, zrows, izeros, dstn, *word_idx, *doc_idx)


def _sc_x_body(gw0, gw1, gw2, xf, segx, zrows, izeros,
               seqsum, acc, nidc, didxc, didx128, b0, b1, m0, m1):
    cid = lax.axis_index("c")
    sid = lax.axis_index("s")
    gws = (gw0, gw1, gw2)
    bufs = (b0, b1)
    sems = (m0, m1)
    rpt = B // NS

    def zero_acc():
        pltpu.sync_copy(zrows.at[pl.ds(0, rpt)], acc.at[pl.ds(sid * rpt, rpt)])

    zero_acc()
    plsc.subcore_barrier()
    gbase = cid * (NCX * 128 * NS) + sid * (NCX * 128)
    pltpu.sync_copy(xf.at[pl.ds(gbase, NCX * 128)],
                    nidc.at[pl.ds(0, NCX * 128)])
    pltpu.sync_copy(segx.at[pl.ds(gbase, NCX * 128)],
                    didxc.at[pl.ds(0, NCX * 128)])
    pltpu.sync_copy(izeros, nidc.at[pl.ds(NCX * 128, 384)])

    def stage_didx(j):
        for k in range(8):
            didx128[pl.ds(k * 16, 16)] = didxc[pl.ds(j * 128 + k * 16, 16)]

    for s_ in range(3):
        tbl = gws[s_]

        def gref(j, tbl=tbl):
            return tbl.at[nidc.at[pl.ds(j * 128, 128)]]

        pltpu.async_copy(gref(0), b0, m0)
        pltpu.async_copy(gref(1), b1, m1)

        def body(j2, _, gref=gref):
            for k in range(2):
                j = j2 * 2 + k
                pltpu.make_async_copy(gref(j), bufs[k], sems[k]).wait()
                stage_didx(j)
                pltpu.sync_copy(bufs[k], acc.at[didx128], add=True)
                pltpu.async_copy(gref(j + 2), bufs[k], sems[k])
            return 0

        lax.fori_loop(0, NCX // 2, body, 0)
        for k in range(2):
            pltpu.make_async_copy(gref(0), bufs[k], sems[k]).wait()
        plsc.subcore_barrier()
        pltpu.sync_copy(acc.at[pl.ds(sid * rpt, rpt)],
                        seqsum.at[s_, cid, pl.ds(sid * rpt, rpt)])
        zero_acc()
        plsc.subcore_barrier()


def _sc_x(gws, xf, segx, zrows, izeros):
    f = pl.kernel(
        _sc_x_body,
        out_type=jax.ShapeDtypeStruct((3, 2, B, HS), jnp.float32),
        mesh=_MESH,
        scratch_types=[
            pltpu.VMEM_SHARED((B + 128, HS), jnp.float32),
            pltpu.VMEM((NCX * 128 + 384,), jnp.int32),
            pltpu.VMEM((NCX * 128,), jnp.int32),
            pltpu.VMEM((128,), jnp.int32),
            pltpu.VMEM((HS, HS), jnp.float32),
            pltpu.VMEM((HS, HS), jnp.float32),
            pltpu.SemaphoreType.DMA,
            pltpu.SemaphoreType.DMA,
        ],
    )
    return f(*gws, xf, segx, zrows, izeros)


def _combine3(h1, h2, h3):
    hs = (h1, h2, h3)
    d = [[jnp.sum(hs[a] * hs[b], axis=1, keepdims=True) * SCALE for b in range(3)]
         for a in range(3)]
    w = [jnp.zeros_like(d[0][0]) for _ in range(3)]
    for a in range(3):
        m = jnp.maximum(jnp.maximum(d[a][0], d[a][1]), d[a][2])
        e = [jnp.exp(d[a][b] - m) for b in range(3)]
        tot = e[0] + e[1] + e[2]
        for b_ in range(3):
            w[b_] = w[b_] + e[b_] / tot
    return (w[0] * h1 + w[1] * h2 + w[2] * h3) * (1.0 / 3.0)


def _word_combine_body(dstf_ref, sums_ref, w1_ref, w2_ref,
                       gw0_ref, gw1_ref, gw2_ref):
    i = pl.program_id(0)
    rows = gw0_ref.shape[0]
    dstf = jnp.concatenate([dstf_ref[0], dstf_ref[1], dstf_ref[2]], axis=-1)
    hs = []
    for r in range(3):
        s = jnp.concatenate([sums_ref[r, 0, 0] + sums_ref[r, 0, 1],
                             sums_ref[r, 1, 0] + sums_ref[r, 1, 1],
                             sums_ref[r, 2, 0] + sums_ref[r, 2, 1]], axis=-1)
        cnt = jnp.maximum(s[:, DP - 1:DP], 1.0)
        mean = s / cnt
        h = (jnp.dot(dstf, w1_ref[r], preferred_element_type=jnp.float32)
             + jnp.dot(mean, w2_ref[r], preferred_element_type=jnp.float32))
        hs.append(h)
    doc = _combine3(*hs)
    row_id = i * rows + lax.broadcasted_iota(jnp.int32, (rows, DP), 0)
    col_id = lax.broadcasted_iota(jnp.int32, (rows, DP), 1)
    base = jnp.where(row_id < ND, doc + dstf, 0.0)
    gwe = jnp.where((col_id == DP - 1) & (row_id <= ND), 1.0, base)
    gw0_ref[...] = gwe[:, :HS]
    gw1_ref[...] = gwe[:, HS:2 * HS]
    gw2_ref[...] = gwe[:, 2 * HS:]


def _word_combine(dstf, sums, w1, w2, rows=512):
    return pl.pallas_call(
        _word_combine_body,
        grid=(NDP // rows,),
        in_specs=[
            pl.BlockSpec((3, rows, HS), lambda i: (0, i, 0)),
            pl.BlockSpec((3, 3, 2, rows, HS), lambda i: (0, 0, 0, i, 0)),
            pl.BlockSpec((3, DP, DP), lambda i: (0, 0, 0)),
            pl.BlockSpec((3, DP, DP), lambda i: (0, 0, 0)),
        ],
        out_specs=[
            pl.BlockSpec((rows, HS), lambda i: (i, 0)),
            pl.BlockSpec((rows, HS), lambda i: (i, 0)),
            pl.BlockSpec((rows, HS), lambda i: (i, 0)),
        ],
        out_shape=[
            jax.ShapeDtypeStruct((NDP, HS), jnp.float32),
            jax.ShapeDtypeStruct((NDP, HS), jnp.float32),
            jax.ShapeDtypeStruct((NDP, HS), jnp.float32),
        ],
    )(dstf, sums, w1, w2)


def _final_body(seqsum_ref, docsums_ref, wd_ref, w1d_ref, w2d_ref, wfc_ref,
                out_ref):
    rows = out_ref.shape[0]
    seqsum = jnp.concatenate([seqsum_ref[0, 0] + seqsum_ref[0, 1],
                              seqsum_ref[1, 0] + seqsum_ref[1, 1],
                              seqsum_ref[2, 0] + seqsum_ref[2, 1]], axis=-1)
    doc_out = jnp.dot(seqsum, wd_ref[...],
                      preferred_element_type=jnp.float32) * (1.0 / L)
    col_id = lax.broadcasted_iota(jnp.int32, (rows, DP), 1)
    dv = jnp.where(col_id == DP - 1, 1.0, doc_out)
    hs = []
    for r in range(3):
        s = jnp.concatenate([docsums_ref[r, 0, 0] + docsums_ref[r, 0, 1],
                             docsums_ref[r, 1, 0] + docsums_ref[r, 1, 1],
                             docsums_ref[r, 2, 0] + docsums_ref[r, 2, 1]],
                            axis=-1)
        cnt = jnp.maximum(s[:, DP - 1:DP], 1.0)
        mean = s / cnt
        h = (jnp.dot(dv, w1d_ref[r], preferred_element_type=jnp.float32)
             + jnp.dot(mean, w2d_ref[r], preferred_element_type=jnp.float32))
        hs.append(h)
    gnn = _combine3(*hs)
    resid = gnn + dv
    out_ref[...] = jnp.dot(resid, wfc_ref[...], preferred_element_type=jnp.float32)


def _final(seqsum, docsums, wd, w1d, w2d, wfc, rows=512):
    return pl.pallas_call(
        _final_body,
        grid=(B // rows,),
        in_specs=[
            pl.BlockSpec((3, 2, rows, HS), lambda i: (0, 0, i, 0)),
            pl.BlockSpec((3, 3, 2, rows, HS), lambda i: (0, 0, 0, i, 0)),
            pl.BlockSpec((DP, DP), lambda i: (0, 0)),
            pl.BlockSpec((3, DP, DP), lambda i: (0, 0, 0)),
            pl.BlockSpec((3, DP, DP), lambda i: (0, 0, 0)),
            pl.BlockSpec((DP, 128), lambda i: (0, 0)),
        ],
        out_specs=pl.BlockSpec((rows, 128), lambda i: (i, 0)),
        out_shape=jax.ShapeDtypeStruct((B, 128), jnp.float32),
    )(seqsum, docsums, wd, w1d, w2d, wfc)


def _pad_w(W, b):
    w1 = jnp.zeros((DP, DP), jnp.float32).at[:D, :D].set(W[:D]).at[DP - 1, :D].set(b)
    w2 = jnp.zeros((DP, DP), jnp.float32).at[:D, :D].set(W[D:])
    return w1, w2


def _slices(emb):
    V = emb.shape[0]
    s2 = jnp.concatenate(
        [emb[:, 2 * HS:D], jnp.zeros((V, DP - 1 - D), jnp.float32),
         jnp.ones((V, 1), jnp.float32)], axis=1)
    return emb[:, :HS], emb[:, HS:2 * HS], s2


def _pad_edges(si, di, n, fill_dst):
    pad = n - si.shape[0]
    si2 = jnp.concatenate([si.astype(jnp.int32), jnp.zeros((pad,), jnp.int32)])
    di2 = jnp.concatenate([di.astype(jnp.int32),
                           jnp.full((pad,), fill_dst, jnp.int32)])
    return si2, di2


def kernel(dst_nids, src_nids_dis, src_nids_pmi, src_nids_top, src_idx_dis, dst_idx_dis, src_idx_pmi, dst_idx_pmi, src_idx_top, dst_idx_top, src_nids_dis_doc, src_nids_pmi_doc, src_nids_top_doc, src_idx_dis_doc, dst_idx_dis_doc, src_idx_pmi_doc, dst_idx_pmi_doc, src_idx_top_doc, dst_idx_top_doc, x_batch, length_batch, return_doc_representation, emb_word, emb_doc, W_dis, b_dis, W_pmi, b_pmi, W_top, b_top, W_dis_d, b_dis_d, W_pmi_d, b_pmi_d, W_top_d, b_top_d, W_dense, b_dense, W_fc, b_fc):
    wts = _slices(emb_word)
    dts = _slices(emb_doc)
    zrows = jnp.zeros((NDP // NS, HS), jnp.float32)
    izeros = jnp.zeros((384,), jnp.int32)
    dstn = jnp.concatenate(
        [dst_nids.astype(jnp.int32), jnp.zeros((DSTP - ND,), jnp.int32)])

    word_idx = []
    for sn, si, di in ((src_nids_dis, src_idx_dis, dst_idx_dis),
                       (src_nids_pmi, src_idx_pmi, dst_idx_pmi),
                       (src_nids_top, src_idx_top, dst_idx_top)):
        word_idx.append(sn.astype(jnp.int32))
    for sn, si, di in ((src_nids_dis, src_idx_dis, dst_idx_dis),
                       (src_nids_pmi, src_idx_pmi, dst_idx_pmi),
                       (src_nids_top, src_idx_top, dst_idx_top)):
        si2, di2 = _pad_edges(si, di, EWP, NDP - 1)
        word_idx.extend([si2, di2])
    doc_idx = []
    for sn, si, di in ((src_nids_dis_doc, src_idx_dis_doc, dst_idx_dis_doc),
                       (src_nids_pmi_doc, src_idx_pmi_doc, dst_idx_pmi_doc),
                       (src_nids_top_doc, src_idx_top_doc, dst_idx_top_doc)):
        doc_idx.append(sn.astype(jnp.int32))
    for sn, si, di in ((src_nids_dis_doc, src_idx_dis_doc, dst_idx_dis_doc),
                       (src_nids_pmi_doc, src_idx_pmi_doc, dst_idx_pmi_doc),
                       (src_nids_top_doc, src_idx_top_doc, dst_idx_top_doc)):
        si2, di2 = _pad_edges(si, di, EDP, B)
        doc_idx.extend([si2, di2])

    sums_w, dstf, sums_d = _sc_a(wts, dts, zrows, izeros, dstn,
                                 word_idx, doc_idx)

    w1s, w2s = [], []
    for W, b_ in ((W_dis, b_dis), (W_pmi, b_pmi), (W_top, b_top)):
        a, b2 = _pad_w(W, b_)
        w1s.append(a)
        w2s.append(b2)
    gws = _word_combine(dstf, sums_w, jnp.stack(w1s), jnp.stack(w2s))

    x_flat = jnp.concatenate([x_batch.reshape(-1).astype(jnp.int32),
                              jnp.full((XFP - B * L,), ND, jnp.int32)])
    seg_x = jnp.concatenate([jnp.repeat(jnp.arange(B, dtype=jnp.int32), L),
                             jnp.full((XFP - B * L,), B, jnp.int32)])
    seqsum = _sc_x(gws, x_flat, seg_x, zrows, izeros)

    wd = jnp.zeros((DP, DP), jnp.float32).at[:D, :D].set(W_dense).at[DP - 1, :D].set(b_dense)
    w1d, w2d = [], []
    for W, b_ in ((W_dis_d, b_dis_d), (W_pmi_d, b_pmi_d), (W_top_d, b_top_d)):
        a, b2 = _pad_w(W, b_)
        w1d.append(a)
        w2d.append(b2)
    wfc = jnp.zeros((DP, 128), jnp.float32).at[:D, :C].set(W_fc).at[DP - 1, :C].set(b_fc)

    out = _final(seqsum, sums_d, wd, jnp.stack(w1d), jnp.stack(w2d), wfc)
    return out[:, :C]


# speedup vs baseline: 1.1507x; 1.1507x over previous
import functools

import jax
import jax.numpy as jnp
from jax import lax
from jax.experimental import pallas as pl
from jax.experimental.pallas import tpu as pltpu
from jax.experimental.pallas import tpu_sc as plsc

D = 300
DP = 384
HS = 128
ND = 10000
NDP = 10240
DSTP = 12288
EW = 160000
EWP = 163840
ED = 16384
B = 1024
L = 200
C = 20
VW = 100000
SCALE = float(D) ** -0.5

NS = 16
NCW = 40
NCD = 8
NCX = 50
EDP = NCD * 128 * 32
XFP = NCX * 128 * 32
CMAX = max(NCW, NCD, NCX) * 128

_MESH = plsc.VectorSubcoreMesh(core_axis_name="c", subcore_axis_name="s")


def _sc_a_body(wt0, wt1, wt2, dt0, dt1, dt2, zrows, izeros, dstn,
               sn_dis, sn_pmi, sn_top,
               si_dis, di_dis, si_pmi, di_pmi, si_top, di_top,
               snd_dis, snd_pmi, snd_top,
               sid_dis, did_dis, sid_pmi, did_pmi, sid_top, did_top,
               sums_w, dstf, sums_d,
               acc, nidc, didxc, didx128, b0, b1, m0, m1):
    cid = lax.axis_index("c")
    sid = lax.axis_index("s")
    wts = (wt0, wt1, wt2)
    dts = (dt0, dt1, dt2)
    bufs = (b0, b1)
    sems = (m0, m1)

    def zero_acc(rpt):
        pltpu.sync_copy(zrows.at[pl.ds(0, rpt)], acc.at[pl.ds(sid * rpt, rpt)])

    def stage_didx(j):
        for k in range(8):
            didx128[pl.ds(k * 16, 16)] = didxc[pl.ds(j * 128 + k * 16, 16)]

    def compose(sn, si1d, di1d, gbase, n):
        pltpu.sync_copy(si1d.at[pl.ds(gbase, n * 128)],
                        didxc.at[pl.ds(0, n * 128)])
        pltpu.sync_copy(izeros, nidc.at[pl.ds(n * 128, 384)])
        descs = [pltpu.async_copy(sn.at[didxc.at[pl.ds(j * 128, 128)]],
                                  nidc.at[pl.ds(j * 128, 128)], m0)
                 for j in range(n)]
        for d_ in descs:
            d_.wait()
        pltpu.sync_copy(di1d.at[pl.ds(gbase, n * 128)],
                        didxc.at[pl.ds(0, n * 128)])

    def load_direct(g1d, d1d, gbase, n):
        pltpu.sync_copy(g1d.at[pl.ds(gbase, n * 128)],
                        nidc.at[pl.ds(0, n * 128)])
        pltpu.sync_copy(d1d.at[pl.ds(gbase, n * 128)],
                        didxc.at[pl.ds(0, n * 128)])
        pltpu.sync_copy(izeros, nidc.at[pl.ds(n * 128, 384)])

    def pipe_scatter(tbl, n):
        def gref(j):
            return tbl.at[nidc.at[pl.ds(j * 128, 128)]]

        pltpu.async_copy(gref(0), b0, m0)
        pltpu.async_copy(gref(1), b1, m1)

        def body(j2, _):
            for k in range(2):
                j = j2 * 2 + k
                pltpu.make_async_copy(gref(j), bufs[k], sems[k]).wait()
                stage_didx(j)
                pltpu.sync_copy(bufs[k], acc.at[didx128], add=True)
                pltpu.async_copy(gref(j + 2), bufs[k], sems[k])
            return 0

        lax.fori_loop(0, n // 2, body, 0)
        for k in range(2):
            pltpu.make_async_copy(gref(0), bufs[k], sems[k]).wait()

    def finish(out_ref, r, s_, rpt):
        plsc.subcore_barrier()
        pltpu.sync_copy(acc.at[pl.ds(sid * rpt, rpt)],
                        out_ref.at[r, s_, cid, pl.ds(sid * rpt, rpt)])
        zero_acc(rpt)
        plsc.subcore_barrier()

    zero_acc(NDP // NS)
    plsc.subcore_barrier()

    word = ((sn_dis, si_dis, di_dis), (sn_pmi, si_pmi, di_pmi),
            (sn_top, si_top, di_top))
    for r, (sn, si1d, di1d) in enumerate(word):
        gbase = cid * (NCW * 128 * NS) + sid * (NCW * 128)
        compose(sn, si1d, di1d, gbase, NCW)
        for s_ in range(3):
            pipe_scatter(wts[s_], NCW)
            finish(sums_w, r, s_, NDP // NS)

    @pl.when(sid < 6)
    def _():
        w = cid * 6 + sid
        pltpu.sync_copy(dstn.at[pl.ds(w * 1024, 1024)],
                        nidc.at[pl.ds(0, 1024)])
        for j in range(8):
            for s_ in range(3):
                pltpu.async_copy(
                    wts[s_].at[nidc.at[pl.ds(j * 128, 128)]], b0, m0
                ).wait()
                pltpu.sync_copy(b0, dstf.at[s_, pl.ds(w * 1024 + j * 128, 128)])

    docr = ((snd_dis, sid_dis, did_dis), (snd_pmi, sid_pmi, did_pmi),
            (snd_top, sid_top, did_top))
    for r, (sn, si1d, di1d) in enumerate(docr):
        gbase = cid * (NCD * 128 * NS) + sid * (NCD * 128)
        compose(sn, si1d, di1d, gbase, NCD)
        for s_ in range(3):
            pipe_scatter(dts[s_], NCD)
            finish(sums_d, r, s_, B // NS)


def _sc_a(wts, dts, zrows, izeros, dstn, word_idx, doc_idx):
    f = pl.kernel(
        _sc_a_body,
        out_type=[
            jax.ShapeDtypeStruct((3, 3, 2, NDP, HS), jnp.float32),
            jax.ShapeDtypeStruct((3, DSTP, HS), jnp.float32),
            jax.ShapeDtypeStruct((3, 3, 2, B, HS), jnp.float32),
        ],
        mesh=_MESH,
        scratch_types=[
            pltpu.VMEM_SHARED((NDP, HS), jnp.float32),
            pltpu.VMEM((NCW * 128 + 384,), jnp.int32),
            pltpu.VMEM((NCW * 128,), jnp.int32),
            pltpu.VMEM((128,), jnp.int32),
            pltpu.VMEM((HS, HS), jnp.float32),
            pltpu.VMEM((HS, HS), jnp.float32),
            pltpu.SemaphoreType.DMA,
            pltpu.SemaphoreType.DMA,
        ],
    )
    return f(*wts, *dts, zrows, izeros, dstn, *word_idx, *doc_idx)


def _sc_x_body(gw0, gw1, gw2, xf, zrows, izeros,
               seqsum, accL, nidc, b0, b1, m0, m1):
    cid = lax.axis_index("c")
    sid = lax.axis_index("s")
    gws = (gw0, gw1, gw2)
    bufs = (b0, b1)
    sems = (m0, m1)
    w = cid * NS + sid
    gbase = w * (NCX * 128)

    pltpu.sync_copy(xf.at[pl.ds(gbase, NCX * 128)],
                    nidc.at[pl.ds(0, NCX * 128)])
    pltpu.sync_copy(izeros, nidc.at[pl.ds(NCX * 128, 384)])
    wbase = w * 32

    for s_ in range(3):
        tbl = gws[s_]
        pltpu.sync_copy(zrows.at[pl.ds(0, 40)], accL)

        def gref(j, tbl=tbl):
            return tbl.at[nidc.at[pl.ds(j * 128, 128)]]

        pltpu.async_copy(gref(0), b0, m0)
        pltpu.async_copy(gref(1), b1, m1)

        def body(j2, _, gref=gref):
            for k in range(2):
                j = j2 * 2 + k
                pltpu.make_async_copy(gref(j), bufs[k], sems[k]).wait()

                def row(i, _, k=k, j=j):
                    d = (j * 128 + i) // 200
                    for q in range(8):
                        plsc.addupdate(accL.at[d, pl.ds(q * 16, 16)],
                                       bufs[k][i, pl.ds(q * 16, 16)])
                    return 0

                lax.fori_loop(0, 128, row, 0)
                pltpu.async_copy(gref(j + 2), bufs[k], sems[k])
            return 0

        lax.fori_loop(0, NCX // 2, body, 0)
        for k in range(2):
            pltpu.make_async_copy(gref(0), bufs[k], sems[k]).wait()
        pltpu.sync_copy(accL.at[pl.ds(0, 32)],
                        seqsum.at[s_, pl.ds(wbase, 32)])


def _sc_x(gws, xf, zrows, izeros):
    f = pl.kernel(
        _sc_x_body,
        out_type=jax.ShapeDtypeStruct((3, B, HS), jnp.float32),
        mesh=_MESH,
        scratch_types=[
            pltpu.VMEM((40, HS), jnp.float32),
            pltpu.VMEM((NCX * 128 + 384,), jnp.int32),
            pltpu.VMEM((HS, HS), jnp.float32),
            pltpu.VMEM((HS, HS), jnp.float32),
            pltpu.SemaphoreType.DMA,
            pltpu.SemaphoreType.DMA,
        ],
    )
    return f(*gws, xf, zrows, izeros)


def _combine3(h1, h2, h3):
    hs = (h1, h2, h3)
    d = [[jnp.sum(hs[a] * hs[b], axis=1, keepdims=True) * SCALE for b in range(3)]
         for a in range(3)]
    w = [jnp.zeros_like(d[0][0]) for _ in range(3)]
    for a in range(3):
        m = jnp.maximum(jnp.maximum(d[a][0], d[a][1]), d[a][2])
        e = [jnp.exp(d[a][b] - m) for b in range(3)]
        tot = e[0] + e[1] + e[2]
        for b_ in range(3):
            w[b_] = w[b_] + e[b_] / tot
    return (w[0] * h1 + w[1] * h2 + w[2] * h3) * (1.0 / 3.0)


def _word_combine_body(dstf_ref, sums_ref, w1_ref, w2_ref,
                       gw0_ref, gw1_ref, gw2_ref):
    i = pl.program_id(0)
    rows = gw0_ref.shape[0]
    dstf = jnp.concatenate([dstf_ref[0], dstf_ref[1], dstf_ref[2]], axis=-1)
    hs = []
    for r in range(3):
        s = jnp.concatenate([sums_ref[r, 0, 0] + sums_ref[r, 0, 1],
                             sums_ref[r, 1, 0] + sums_ref[r, 1, 1],
                             sums_ref[r, 2, 0] + sums_ref[r, 2, 1]], axis=-1)
        cnt = jnp.maximum(s[:, DP - 1:DP], 1.0)
        mean = s / cnt
        h = (jnp.dot(dstf, w1_ref[r], preferred_element_type=jnp.float32)
             + jnp.dot(mean, w2_ref[r], preferred_element_type=jnp.float32))
        hs.append(h)
    doc = _combine3(*hs)
    row_id = i * rows + lax.broadcasted_iota(jnp.int32, (rows, DP), 0)
    col_id = lax.broadcasted_iota(jnp.int32, (rows, DP), 1)
    base = jnp.where(row_id < ND, doc + dstf, 0.0)
    gwe = jnp.where((col_id == DP - 1) & (row_id <= ND), 1.0, base)
    gw0_ref[...] = gwe[:, :HS]
    gw1_ref[...] = gwe[:, HS:2 * HS]
    gw2_ref[...] = gwe[:, 2 * HS:]


def _word_combine(dstf, sums, w1, w2, rows=512):
    return pl.pallas_call(
        _word_combine_body,
        grid=(NDP // rows,),
        in_specs=[
            pl.BlockSpec((3, rows, HS), lambda i: (0, i, 0)),
            pl.BlockSpec((3, 3, 2, rows, HS), lambda i: (0, 0, 0, i, 0)),
            pl.BlockSpec((3, DP, DP), lambda i: (0, 0, 0)),
            pl.BlockSpec((3, DP, DP), lambda i: (0, 0, 0)),
        ],
        out_specs=[
            pl.BlockSpec((rows, HS), lambda i: (i, 0)),
            pl.BlockSpec((rows, HS), lambda i: (i, 0)),
            pl.BlockSpec((rows, HS), lambda i: (i, 0)),
        ],
        out_shape=[
            jax.ShapeDtypeStruct((NDP, HS), jnp.float32),
            jax.ShapeDtypeStruct((NDP, HS), jnp.float32),
            jax.ShapeDtypeStruct((NDP, HS), jnp.float32),
        ],
    )(dstf, sums, w1, w2)


def _final_body(seqsum_ref, docsums_ref, wd_ref, w1d_ref, w2d_ref, wfc_ref,
                out_ref):
    rows = out_ref.shape[0]
    seqsum = jnp.concatenate([seqsum_ref[0], seqsum_ref[1],
                              seqsum_ref[2]], axis=-1)
    doc_out = jnp.dot(seqsum, wd_ref[...],
                      preferred_element_type=jnp.float32) * (1.0 / L)
    col_id = lax.broadcasted_iota(jnp.int32, (rows, DP), 1)
    dv = jnp.where(col_id == DP - 1, 1.0, doc_out)
    hs = []
    for r in range(3):
        s = jnp.concatenate([docsums_ref[r, 0, 0] + docsums_ref[r, 0, 1],
                             docsums_ref[r, 1, 0] + docsums_ref[r, 1, 1],
                             docsums_ref[r, 2, 0] + docsums_ref[r, 2, 1]],
                            axis=-1)
        cnt = jnp.maximum(s[:, DP - 1:DP], 1.0)
        mean = s / cnt
        h = (jnp.dot(dv, w1d_ref[r], preferred_element_type=jnp.float32)
             + jnp.dot(mean, w2d_ref[r], preferred_element_type=jnp.float32))
        hs.append(h)
    gnn = _combine3(*hs)
    resid = gnn + dv
    out_ref[...] = jnp.dot(resid, wfc_ref[...], preferred_element_type=jnp.float32)


def _final(seqsum, docsums, wd, w1d, w2d, wfc, rows=512):
    return pl.pallas_call(
        _final_body,
        grid=(B // rows,),
        in_specs=[
            pl.BlockSpec((3, rows, HS), lambda i: (0, i, 0)),
            pl.BlockSpec((3, 3, 2, rows, HS), lambda i: (0, 0, 0, i, 0)),
            pl.BlockSpec((DP, DP), lambda i: (0, 0)),
            pl.BlockSpec((3, DP, DP), lambda i: (0, 0, 0)),
            pl.BlockSpec((3, DP, DP), lambda i: (0, 0, 0)),
            pl.BlockSpec((DP, 128), lambda i: (0, 0)),
        ],
        out_specs=pl.BlockSpec((rows, 128), lambda i: (i, 0)),
        out_shape=jax.ShapeDtypeStruct((B, 128), jnp.float32),
    )(seqsum, docsums, wd, w1d, w2d, wfc)


def _pad_w(W, b):
    w1 = jnp.zeros((DP, DP), jnp.float32).at[:D, :D].set(W[:D]).at[DP - 1, :D].set(b)
    w2 = jnp.zeros((DP, DP), jnp.float32).at[:D, :D].set(W[D:])
    return w1, w2


def _slices(emb):
    V = emb.shape[0]
    s2 = jnp.concatenate(
        [emb[:, 2 * HS:D], jnp.zeros((V, DP - 1 - D), jnp.float32),
         jnp.ones((V, 1), jnp.float32)], axis=1)
    return emb[:, :HS], emb[:, HS:2 * HS], s2


def _pad_edges(si, di, n, fill_dst):
    pad = n - si.shape[0]
    si2 = jnp.concatenate([si.astype(jnp.int32), jnp.zeros((pad,), jnp.int32)])
    di2 = jnp.concatenate([di.astype(jnp.int32),
                           jnp.full((pad,), fill_dst, jnp.int32)])
    return si2, di2


def kernel(dst_nids, src_nids_dis, src_nids_pmi, src_nids_top, src_idx_dis, dst_idx_dis, src_idx_pmi, dst_idx_pmi, src_idx_top, dst_idx_top, src_nids_dis_doc, src_nids_pmi_doc, src_nids_top_doc, src_idx_dis_doc, dst_idx_dis_doc, src_idx_pmi_doc, dst_idx_pmi_doc, src_idx_top_doc, dst_idx_top_doc, x_batch, length_batch, return_doc_representation, emb_word, emb_doc, W_dis, b_dis, W_pmi, b_pmi, W_top, b_top, W_dis_d, b_dis_d, W_pmi_d, b_pmi_d, W_top_d, b_top_d, W_dense, b_dense, W_fc, b_fc):
    wts = _slices(emb_word)
    dts = _slices(emb_doc)
    zrows = jnp.zeros((NDP // NS, HS), jnp.float32)
    izeros = jnp.zeros((384,), jnp.int32)
    dstn = jnp.concatenate(
        [dst_nids.astype(jnp.int32), jnp.zeros((DSTP - ND,), jnp.int32)])

    word_idx = []
    for sn, si, di in ((src_nids_dis, src_idx_dis, dst_idx_dis),
                       (src_nids_pmi, src_idx_pmi, dst_idx_pmi),
                       (src_nids_top, src_idx_top, dst_idx_top)):
        word_idx.append(sn.astype(jnp.int32))
    for sn, si, di in ((src_nids_dis, src_idx_dis, dst_idx_dis),
                       (src_nids_pmi, src_idx_pmi, dst_idx_pmi),
                       (src_nids_top, src_idx_top, dst_idx_top)):
        si2, di2 = _pad_edges(si, di, EWP, NDP - 1)
        word_idx.extend([si2, di2])
    doc_idx = []
    for sn, si, di in ((src_nids_dis_doc, src_idx_dis_doc, dst_idx_dis_doc),
                       (src_nids_pmi_doc, src_idx_pmi_doc, dst_idx_pmi_doc),
                       (src_nids_top_doc, src_idx_top_doc, dst_idx_top_doc)):
        doc_idx.append(sn.astype(jnp.int32))
    for sn, si, di in ((src_nids_dis_doc, src_idx_dis_doc, dst_idx_dis_doc),
                       (src_nids_pmi_doc, src_idx_pmi_doc, dst_idx_pmi_doc),
                       (src_nids_top_doc, src_idx_top_doc, dst_idx_top_doc)):
        si2, di2 = _pad_edges(si, di, EDP, B)
        doc_idx.extend([si2, di2])

    sums_w, dstf, sums_d = _sc_a(wts, dts, zrows, izeros, dstn,
                                 word_idx, doc_idx)

    w1s, w2s = [], []
    for W, b_ in ((W_dis, b_dis), (W_pmi, b_pmi), (W_top, b_top)):
        a, b2 = _pad_w(W, b_)
        w1s.append(a)
        w2s.append(b2)
    gws = _word_combine(dstf, sums_w, jnp.stack(w1s), jnp.stack(w2s))

    x_flat = x_batch.reshape(-1).astype(jnp.int32)
    seqsum = _sc_x(gws, x_flat, zrows, izeros)

    wd = jnp.zeros((DP, DP), jnp.float32).at[:D, :D].set(W_dense).at[DP - 1, :D].set(b_dense)
    w1d, w2d = [], []
    for W, b_ in ((W_dis_d, b_dis_d), (W_pmi_d, b_pmi_d), (W_top_d, b_top_d)):
        a, b2 = _pad_w(W, b_)
        w1d.append(a)
        w2d.append(b2)
    wfc = jnp.zeros((DP, 128), jnp.float32).at[:D, :C].set(W_fc).at[DP - 1, :C].set(b_fc)

    out = _final(seqsum, sums_d, wd, jnp.stack(w1d), jnp.stack(w2d), wfc)
    return out[:, :C]


# speedup vs baseline: 1.1526x; 1.0017x over previous
import jax
import jax.numpy as jnp
from jax import lax
from jax.experimental import pallas as pl
from jax.experimental.pallas import tpu as pltpu
from jax.experimental.pallas import tpu_sc as plsc

D = 300
DP = 384
HS = 128
ND = 10000
NDP = 10240
DSTP = 12288
EW = 160000
EWP = 163840
ED = 16384
B = 1024
L = 200
C = 20
VW = 100000
SCALE = float(D) ** -0.5

NS = 16
NCW = 40
NCD = 8
NCX = 50
EDP = NCD * 128 * 32

_MESH = plsc.VectorSubcoreMesh(core_axis_name="c", subcore_axis_name="s")


def _sc_a_body(wt0, wt1, wt2, dt0, dt1, dt2, zrows, izeros, dstn,
               sn_dis, sn_pmi, sn_top,
               si_dis, di_dis, si_pmi, di_pmi, si_top, di_top,
               snd_dis, snd_pmi, snd_top,
               sid_dis, did_dis, sid_pmi, did_pmi, sid_top, did_top,
               sums_w, dstf, sums_d,
               acc, nidc, didxc, didx128, b0, b1, m0, m1):
    cid = lax.axis_index("c")
    sid = lax.axis_index("s")
    wts = (wt0, wt1, wt2)
    dts = (dt0, dt1, dt2)
    bufs = (b0, b1)
    sems = (m0, m1)

    def zero_acc(rpt):
        pltpu.sync_copy(zrows.at[pl.ds(0, rpt)], acc.at[pl.ds(sid * rpt, rpt)])

    def stage_didx(j):
        for k in range(8):
            didx128[pl.ds(k * 16, 16)] = didxc[pl.ds(j * 128 + k * 16, 16)]

    def compose(sn, si1d, di1d, gbase, n):
        pltpu.sync_copy(si1d.at[pl.ds(gbase, n * 128)],
                        didxc.at[pl.ds(0, n * 128)])
        pltpu.sync_copy(izeros, nidc.at[pl.ds(n * 128, 384)])
        descs = [pltpu.async_copy(sn.at[didxc.at[pl.ds(j * 128, 128)]],
                                  nidc.at[pl.ds(j * 128, 128)], m0)
                 for j in range(n)]
        for d_ in descs:
            d_.wait()
        pltpu.sync_copy(di1d.at[pl.ds(gbase, n * 128)],
                        didxc.at[pl.ds(0, n * 128)])

    def pipe_scatter(tbl, n):
        def gref(j):
            return tbl.at[nidc.at[pl.ds(j * 128, 128)]]

        pltpu.async_copy(gref(0), b0, m0)
        pltpu.async_copy(gref(1), b1, m1)

        def body(j2, _):
            for k in range(2):
                j = j2 * 2 + k
                pltpu.make_async_copy(gref(j), bufs[k], sems[k]).wait()
                stage_didx(j)
                pltpu.sync_copy(bufs[k], acc.at[didx128], add=True)
                pltpu.async_copy(gref(j + 2), bufs[k], sems[k])
            return 0

        lax.fori_loop(0, n // 2, body, 0)
        for k in range(2):
            pltpu.make_async_copy(gref(0), bufs[k], sems[k]).wait()

    def finish(out_ref, r, s_, rpt):
        plsc.subcore_barrier()
        pltpu.sync_copy(acc.at[pl.ds(sid * rpt, rpt)],
                        out_ref.at[r, s_, cid, pl.ds(sid * rpt, rpt)])
        zero_acc(rpt)
        plsc.subcore_barrier()

    zero_acc(NDP // NS)
    plsc.subcore_barrier()

    word = ((sn_dis, si_dis, di_dis), (sn_pmi, si_pmi, di_pmi),
            (sn_top, si_top, di_top))
    for r, (sn, si1d, di1d) in enumerate(word):
        gbase = cid * (NCW * 128 * NS) + sid * (NCW * 128)
        compose(sn, si1d, di1d, gbase, NCW)
        for s_ in range(3):
            pipe_scatter(wts[s_], NCW)
            finish(sums_w, r, s_, NDP // NS)

    @pl.when(sid < 6)
    def _():
        w = cid * 6 + sid
        pltpu.sync_copy(dstn.at[pl.ds(w * 1024, 1024)],
                        nidc.at[pl.ds(0, 1024)])
        for j in range(8):
            for s_ in range(3):
                pltpu.async_copy(
                    wts[s_].at[nidc.at[pl.ds(j * 128, 128)]], b0, m0
                ).wait()
                pltpu.sync_copy(b0, dstf.at[s_, pl.ds(w * 1024 + j * 128, 128)])

    docr = ((snd_dis, sid_dis, did_dis), (snd_pmi, sid_pmi, did_pmi),
            (snd_top, sid_top, did_top))
    for r, (sn, si1d, di1d) in enumerate(docr):
        gbase = cid * (NCD * 128 * NS) + sid * (NCD * 128)
        compose(sn, si1d, di1d, gbase, NCD)
        for s_ in range(3):
            pipe_scatter(dts[s_], NCD)
            finish(sums_d, r, s_, B // NS)


def _sc_a(wts, dts, zrows, izeros, dstn, word_idx, doc_idx):
    f = pl.kernel(
        _sc_a_body,
        out_type=[
            jax.ShapeDtypeStruct((3, 3, 2, NDP, HS), jnp.float32),
            jax.ShapeDtypeStruct((3, DSTP, HS), jnp.float32),
            jax.ShapeDtypeStruct((3, 3, 2, B, HS), jnp.float32),
        ],
        mesh=_MESH,
        scratch_types=[
            pltpu.VMEM_SHARED((NDP, HS), jnp.float32),
            pltpu.VMEM((NCW * 128 + 384,), jnp.int32),
            pltpu.VMEM((NCW * 128,), jnp.int32),
            pltpu.VMEM((128,), jnp.int32),
            pltpu.VMEM((HS, HS), jnp.float32),
            pltpu.VMEM((HS, HS), jnp.float32),
            pltpu.SemaphoreType.DMA,
            pltpu.SemaphoreType.DMA,
        ],
    )
    return f(*wts, *dts, zrows, izeros, dstn, *word_idx, *doc_idx)


def _sc_x_body(gw0, gw1, gw2, xf, zrows, izeros,
               seqsum, accL, nidc, b0, b1, m0, m1):
    cid = lax.axis_index("c")
    sid = lax.axis_index("s")
    gws = (gw0, gw1, gw2)
    bufs = (b0, b1)
    sems = (m0, m1)
    w = cid * NS + sid
    gbase = w * (NCX * 128)

    pltpu.sync_copy(xf.at[pl.ds(gbase, NCX * 128)],
                    nidc.at[pl.ds(0, NCX * 128)])
    pltpu.sync_copy(izeros, nidc.at[pl.ds(NCX * 128, 384)])
    wbase = w * 32

    for s_ in range(3):
        tbl = gws[s_]
        pltpu.sync_copy(zrows.at[pl.ds(0, 40)], accL)

        def gref(j, tbl=tbl):
            return tbl.at[nidc.at[pl.ds(j * 128, 128)]]

        pltpu.async_copy(gref(0), b0, m0)
        pltpu.async_copy(gref(1), b1, m1)

        def body(j2, _, gref=gref):
            for k in range(2):
                j = j2 * 2 + k
                pltpu.make_async_copy(gref(j), bufs[k], sems[k]).wait()

                def row(i, _, k=k, j=j):
                    d = (j * 128 + i) // 200
                    for q in range(8):
                        plsc.addupdate(accL.at[d, pl.ds(q * 16, 16)],
                                       bufs[k][i, pl.ds(q * 16, 16)])
                    return 0

                lax.fori_loop(0, 128, row, 0)
                pltpu.async_copy(gref(j + 2), bufs[k], sems[k])
            return 0

        lax.fori_loop(0, NCX // 2, body, 0)
        for k in range(2):
            pltpu.make_async_copy(gref(0), bufs[k], sems[k]).wait()
        pltpu.sync_copy(accL.at[pl.ds(0, 32)],
                        seqsum.at[s_, pl.ds(wbase, 32)])


def _sc_x(gws, xf, zrows, izeros):
    f = pl.kernel(
        _sc_x_body,
        out_type=jax.ShapeDtypeStruct((3, B, HS), jnp.float32),
        mesh=_MESH,
        scratch_types=[
            pltpu.VMEM((40, HS), jnp.float32),
            pltpu.VMEM((NCX * 128 + 384,), jnp.int32),
            pltpu.VMEM((HS, HS), jnp.float32),
            pltpu.VMEM((HS, HS), jnp.float32),
            pltpu.SemaphoreType.DMA,
            pltpu.SemaphoreType.DMA,
        ],
    )
    return f(*gws, xf, zrows, izeros)


def _combine3(h1, h2, h3):
    hs = (h1, h2, h3)
    d = [[jnp.sum(hs[a] * hs[b], axis=1, keepdims=True) * SCALE for b in range(3)]
         for a in range(3)]
    w = [jnp.zeros_like(d[0][0]) for _ in range(3)]
    for a in range(3):
        m = jnp.maximum(jnp.maximum(d[a][0], d[a][1]), d[a][2])
        e = [jnp.exp(d[a][b] - m) for b in range(3)]
        tot = e[0] + e[1] + e[2]
        for b_ in range(3):
            w[b_] = w[b_] + e[b_] / tot
    return (w[0] * h1 + w[1] * h2 + w[2] * h3) * (1.0 / 3.0)


def _word_combine_body(dstf_ref, sums_ref, w1_ref, w2_ref,
                       gw0_ref, gw1_ref, gw2_ref):
    i = pl.program_id(0)
    rows = gw0_ref.shape[0]
    dstf = jnp.concatenate([dstf_ref[0], dstf_ref[1], dstf_ref[2]], axis=-1)
    hs = []
    for r in range(3):
        s = jnp.concatenate([sums_ref[r, 0, 0] + sums_ref[r, 0, 1],
                             sums_ref[r, 1, 0] + sums_ref[r, 1, 1],
                             sums_ref[r, 2, 0] + sums_ref[r, 2, 1]], axis=-1)
        cnt = jnp.maximum(s[:, DP - 1:DP], 1.0)
        mean = s / cnt
        h = (jnp.dot(dstf, w1_ref[r], preferred_element_type=jnp.float32)
             + jnp.dot(mean, w2_ref[r], preferred_element_type=jnp.float32))
        hs.append(h)
    doc = _combine3(*hs)
    row_id = i * rows + lax.broadcasted_iota(jnp.int32, (rows, DP), 0)
    col_id = lax.broadcasted_iota(jnp.int32, (rows, DP), 1)
    base = jnp.where(row_id < ND, doc + dstf, 0.0)
    gwe = jnp.where((col_id == DP - 1) & (row_id <= ND), 1.0, base)
    gw0_ref[...] = gwe[:, :HS]
    gw1_ref[...] = gwe[:, HS:2 * HS]
    gw2_ref[...] = gwe[:, 2 * HS:]


def _word_combine(dstf, sums, w1, w2, rows=512):
    return pl.pallas_call(
        _word_combine_body,
        grid=(NDP // rows,),
        in_specs=[
            pl.BlockSpec((3, rows, HS), lambda i: (0, i, 0)),
            pl.BlockSpec((3, 3, 2, rows, HS), lambda i: (0, 0, 0, i, 0)),
            pl.BlockSpec((3, DP, DP), lambda i: (0, 0, 0)),
            pl.BlockSpec((3, DP, DP), lambda i: (0, 0, 0)),
        ],
        out_specs=[
            pl.BlockSpec((rows, HS), lambda i: (i, 0)),
            pl.BlockSpec((rows, HS), lambda i: (i, 0)),
            pl.BlockSpec((rows, HS), lambda i: (i, 0)),
        ],
        out_shape=[
            jax.ShapeDtypeStruct((NDP, HS), jnp.float32),
            jax.ShapeDtypeStruct((NDP, HS), jnp.float32),
            jax.ShapeDtypeStruct((NDP, HS), jnp.float32),
        ],
    )(dstf, sums, w1, w2)


def _final_body(seqsum_ref, docsums_ref, wd_ref, w1d_ref, w2d_ref, wfc_ref,
                out_ref):
    rows = out_ref.shape[0]
    seqsum = jnp.concatenate([seqsum_ref[0], seqsum_ref[1],
                              seqsum_ref[2]], axis=-1)
    doc_out = jnp.dot(seqsum, wd_ref[...],
                      preferred_element_type=jnp.float32) * (1.0 / L)
    col_id = lax.broadcasted_iota(jnp.int32, (rows, DP), 1)
    dv = jnp.where(col_id == DP - 1, 1.0, doc_out)
    hs = []
    for r in range(3):
        s = jnp.concatenate([docsums_ref[r, 0, 0] + docsums_ref[r, 0, 1],
                             docsums_ref[r, 1, 0] + docsums_ref[r, 1, 1],
                             docsums_ref[r, 2, 0] + docsums_ref[r, 2, 1]],
                            axis=-1)
        cnt = jnp.maximum(s[:, DP - 1:DP], 1.0)
        mean = s / cnt
        h = (jnp.dot(dv, w1d_ref[r], preferred_element_type=jnp.float32)
             + jnp.dot(mean, w2d_ref[r], preferred_element_type=jnp.float32))
        hs.append(h)
    gnn = _combine3(*hs)
    resid = gnn + dv
    out_ref[...] = jnp.dot(resid, wfc_ref[...], preferred_element_type=jnp.float32)


def _final(seqsum, docsums, wd, w1d, w2d, wfc, rows=512):
    return pl.pallas_call(
        _final_body,
        grid=(B // rows,),
        in_specs=[
            pl.BlockSpec((3, rows, HS), lambda i: (0, i, 0)),
            pl.BlockSpec((3, 3, 2, rows, HS), lambda i: (0, 0, 0, i, 0)),
            pl.BlockSpec((DP, DP), lambda i: (0, 0)),
            pl.BlockSpec((3, DP, DP), lambda i: (0, 0, 0)),
            pl.BlockSpec((3, DP, DP), lambda i: (0, 0, 0)),
            pl.BlockSpec((DP, 128), lambda i: (0, 0)),
        ],
        out_specs=pl.BlockSpec((rows, 128), lambda i: (i, 0)),
        out_shape=jax.ShapeDtypeStruct((B, 128), jnp.float32),
    )(seqsum, docsums, wd, w1d, w2d, wfc)


def _pad_w(W, b):
    w1 = jnp.zeros((DP, DP), jnp.float32).at[:D, :D].set(W[:D]).at[DP - 1, :D].set(b)
    w2 = jnp.zeros((DP, DP), jnp.float32).at[:D, :D].set(W[D:])
    return w1, w2


def _slices(emb):
    V = emb.shape[0]
    s2 = jnp.concatenate(
        [emb[:, 2 * HS:D], jnp.zeros((V, DP - 1 - D), jnp.float32),
         jnp.ones((V, 1), jnp.float32)], axis=1)
    return emb[:, :HS], emb[:, HS:2 * HS], s2


def _pad_edges(si, di, n, fill_dst):
    pad = n - si.shape[0]
    si2 = jnp.concatenate([si.astype(jnp.int32), jnp.zeros((pad,), jnp.int32)])
    di2 = jnp.concatenate([di.astype(jnp.int32),
                           jnp.full((pad,), fill_dst, jnp.int32)])
    return si2, di2


def kernel(dst_nids, src_nids_dis, src_nids_pmi, src_nids_top, src_idx_dis, dst_idx_dis, src_idx_pmi, dst_idx_pmi, src_idx_top, dst_idx_top, src_nids_dis_doc, src_nids_pmi_doc, src_nids_top_doc, src_idx_dis_doc, dst_idx_dis_doc, src_idx_pmi_doc, dst_idx_pmi_doc, src_idx_top_doc, dst_idx_top_doc, x_batch, length_batch, return_doc_representation, emb_word, emb_doc, W_dis, b_dis, W_pmi, b_pmi, W_top, b_top, W_dis_d, b_dis_d, W_pmi_d, b_pmi_d, W_top_d, b_top_d, W_dense, b_dense, W_fc, b_fc):
    wts = _slices(emb_word)
    dts = _slices(emb_doc)
    zrows = jnp.zeros((NDP // NS, HS), jnp.float32)
    izeros = jnp.zeros((384,), jnp.int32)
    dstn = jnp.concatenate(
        [dst_nids.astype(jnp.int32), jnp.zeros((DSTP - ND,), jnp.int32)])

    word_idx = [src_nids_dis.astype(jnp.int32),
                src_nids_pmi.astype(jnp.int32),
                src_nids_top.astype(jnp.int32)]
    for si, di in ((src_idx_dis, dst_idx_dis), (src_idx_pmi, dst_idx_pmi),
                   (src_idx_top, dst_idx_top)):
        si2, di2 = _pad_edges(si, di, EWP, NDP - 1)
        word_idx.extend([si2, di2])
    doc_idx = [src_nids_dis_doc.astype(jnp.int32),
               src_nids_pmi_doc.astype(jnp.int32),
               src_nids_top_doc.astype(jnp.int32)]
    for si, di in ((src_idx_dis_doc, dst_idx_dis_doc),
                   (src_idx_pmi_doc, dst_idx_pmi_doc),
                   (src_idx_top_doc, dst_idx_top_doc)):
        si2, di2 = _pad_edges(si, di, EDP, B)
        doc_idx.extend([si2, di2])

    sums_w, dstf, sums_d = _sc_a(wts, dts, zrows, izeros, dstn,
                                 word_idx, doc_idx)

    w1s, w2s = [], []
    for W, b_ in ((W_dis, b_dis), (W_pmi, b_pmi), (W_top, b_top)):
        a, b2 = _pad_w(W, b_)
        w1s.append(a)
        w2s.append(b2)
    gws = _word_combine(dstf, sums_w, jnp.stack(w1s), jnp.stack(w2s))

    x_flat = x_batch.reshape(-1).astype(jnp.int32)
    seqsum = _sc_x(gws, x_flat, zrows, izeros)

    wd = jnp.zeros((DP, DP), jnp.float32).at[:D, :D].set(W_dense).at[DP - 1, :D].set(b_dense)
    w1d, w2d = [], []
    for W, b_ in ((W_dis_d, b_dis_d), (W_pmi_d, b_pmi_d), (W_top_d, b_top_d)):
        a, b2 = _pad_w(W, b_)
        w1d.append(a)
        w2d.append(b2)
    wfc = jnp.zeros((DP, 128), jnp.float32).at[:D, :C].set(W_fc).at[DP - 1, :C].set(b_fc)

    out = _final(seqsum, sums_d, wd, jnp.stack(w1d), jnp.stack(w2d), wfc)
    return out[:, :C]
